# Initial kernel scaffold; baseline (speedup 1.0000x reference)
import jax
import jax.numpy as jnp
from jax.experimental import pallas as pl


def kernel(x, tables):
    raise NotImplementedError("write your pallas kernel here")


# speedup vs baseline: 1.1191x; 1.1191x over previous
import functools

import jax
import jax.numpy as jnp
from jax import lax
from jax.experimental import pallas as pl
from jax.experimental.pallas import tpu as pltpu
from jax.experimental.pallas import tpu_sc as plsc

NUM_FIELDS = 26
VOCAB = 100000
EMB_DIM = 32
BATCH = 16384

N_ROWS = BATCH * NUM_FIELDS
NW = 32
LANES = 16
SUB = 128
ROWS_PER_W = N_ROWS // NW
N_SUBROWS = N_ROWS // SUB
SUBROWS_PER_W = N_SUBROWS // NW
CHUNK_SUBROWS = 13
N_CHUNKS = SUBROWS_PER_W // CHUNK_SUBROWS
CHUNK_ROWS = CHUNK_SUBROWS * SUB

_mesh = plsc.VectorSubcoreMesh(core_axis_name="c", subcore_axis_name="s")


@functools.partial(
    pl.kernel,
    out_type=jax.ShapeDtypeStruct((N_SUBROWS, SUB, EMB_DIM), jnp.float32),
    mesh=_mesh,
    scratch_types=[
        pltpu.VMEM((CHUNK_SUBROWS, SUB), jnp.int32),
        pltpu.VMEM((CHUNK_SUBROWS, SUB), jnp.int32),
        pltpu.VMEM((CHUNK_SUBROWS, SUB, EMB_DIM), jnp.float32),
        pltpu.SemaphoreType.DMA,
    ],
    compiler_params=pltpu.CompilerParams(use_tc_tiling_on_sc=False),
)
def _gather_kernel(tables_hbm, x_hbm, out_hbm, offs_v, idx_v, rows_v, sem):
    wid = lax.axis_index("s") * 2 + lax.axis_index("c")
    row_base = wid * SUBROWS_PER_W

    def _init_body(i, carry):
        r = i // (SUB // LANES)
        k = i % (SUB // LANES)
        j0 = r * SUB + k * LANES
        vals = lax.rem(lax.iota(jnp.int32, LANES) + j0, NUM_FIELDS) * VOCAB
        offs_v[r, pl.ds(k * LANES, LANES)] = vals
        return carry

    lax.fori_loop(0, CHUNK_SUBROWS * (SUB // LANES), _init_body, 0)

    def _chunk_body(c, carry):
        row = row_base + c * CHUNK_SUBROWS
        pltpu.sync_copy(x_hbm.at[pl.ds(row, CHUNK_SUBROWS)], idx_v)

        def _add_body(i, carry2):
            r = i // (SUB // LANES)
            k = i % (SUB // LANES)
            sl = pl.ds(k * LANES, LANES)
            idx_v[r, sl] = idx_v[r, sl] + offs_v[r, sl]
            return carry2

        lax.fori_loop(0, CHUNK_SUBROWS * (SUB // LANES), _add_body, 0)

        descs = [
            pltpu.async_copy(tables_hbm.at[idx_v.at[j]], rows_v.at[j], sem)
            for j in range(CHUNK_SUBROWS)
        ]
        for d in descs:
            d.wait()

        pltpu.sync_copy(rows_v, out_hbm.at[pl.ds(row, CHUNK_SUBROWS)])
        return carry

    lax.fori_loop(0, N_CHUNKS, _chunk_body, 0)


def kernel(x, tables):
    x_flat = x.astype(jnp.int32).reshape(N_SUBROWS, SUB)
    tables_flat = tables.reshape(NUM_FIELDS * VOCAB, EMB_DIM)
    out = _gather_kernel(tables_flat, x_flat)
    return out.reshape(BATCH, NUM_FIELDS * EMB_DIM)
